# baseline (speedup 1.0000x reference)
import functools

import jax
import jax.numpy as jnp
from jax import lax
from jax.experimental import pallas as pl
from jax.experimental.pallas import tpu as pltpu
from jax.experimental.pallas import tpu_sc as plsc

EMB = 50
HPAD = 56
WPAIR = 2 * HPAD
NC, NS = 2, 16
NW = NC * NS
IDX_TILE = 128
PTILES = 2
PAIRS = IDX_TILE * PTILES
CHUNK = 2 * PAIRS
NBUF = 2
L = 16


@functools.cache
def _make(B):
    assert B % (NW * CHUNK * NBUF) == 0
    b_per_w = B // NW
    n_chunks = b_per_w // CHUNK
    mesh = plsc.VectorSubcoreMesh(core_axis_name="c", subcore_axis_name="s")

    @functools.partial(
        pl.kernel,
        mesh=mesh,
        out_type=jax.ShapeDtypeStruct((B, EMB), jnp.float32),
        compiler_params=pltpu.CompilerParams(
            use_tc_tiling_on_sc=False, needs_layout_passes=False),
        scratch_types=[
            pltpu.VMEM((NBUF, CHUNK), jnp.int32),
            pltpu.VMEM((NBUF, PTILES, IDX_TILE), jnp.int32),
            pltpu.VMEM((NBUF, PAIRS, WPAIR), jnp.float32),
            pltpu.VMEM((NBUF, CHUNK, EMB), jnp.float32),
            pltpu.SemaphoreType.DMA,
            pltpu.SemaphoreType.DMA,
        ],
    )
    def k(batch_hbm, ptable_hbm, out_hbm, idx_v, pidx_v, rows_v, dense_v,
          sem_g, sem_o):
        s = lax.axis_index("s")
        wid = s * NC + lax.axis_index("c")
        w0 = wid * b_per_w
        ptable_sh = ptable_hbm

        def fire_gathers(i, b):
            pltpu.sync_copy(batch_hbm.at[pl.ds(w0 + i * CHUNK, CHUNK)],
                            idx_v.at[b])

            for j in range(PTILES):
                @plsc.parallel_loop(0, IDX_TILE, step=L, unroll=4)
                def _pair(q, j=j):
                    pos = 2 * (j * IDX_TILE + q) + 2 * lax.iota(jnp.int32, L)
                    a = plsc.load_gather(idx_v.at[b], [pos])
                    c = plsc.load_gather(idx_v.at[b], [pos + 1])
                    pidx_v[b, j, pl.ds(q, L)] = a * 128 + c

            for j in range(PTILES):
                pltpu.make_async_copy(
                    ptable_sh.at[pidx_v.at[b].at[j]],
                    rows_v.at[b].at[pl.ds(j * IDX_TILE, IDX_TILE)],
                    sem_g,
                ).start()

        def wait_gathers(b):
            for j in range(PTILES):
                pltpu.make_async_copy(
                    ptable_sh.at[pidx_v.at[b].at[j]],
                    rows_v.at[b].at[pl.ds(j * IDX_TILE, IDX_TILE)],
                    sem_g,
                ).wait()

        def out_copy(i, b):
            return pltpu.make_async_copy(
                dense_v.at[b], out_hbm.at[pl.ds(w0 + i * CHUNK, CHUNK)], sem_o)

        def process(i, b):
            wait_gathers(b)

            @plsc.parallel_loop(0, PAIRS, unroll=4)
            def _row(r):
                for off in (0, 16, 32, 34):
                    dense_v[b, 2 * r, pl.ds(off, L)] = (
                        rows_v[b, r, pl.ds(off, L)])
                for off in (0, 16, 32, 34):
                    dense_v[b, 2 * r + 1, pl.ds(off, L)] = (
                        rows_v[b, r, pl.ds(HPAD + off, L)])

            out_copy(i, b).start()

        fire_gathers(0, 0)

        def step(t, carry):
            i0 = t * 2

            fire_gathers(i0 + 1, 1)

            @pl.when(t > 0)
            def _():
                out_copy(i0 - 2, 0).wait()

            process(i0, 0)

            @pl.when(i0 + 2 < n_chunks)
            def _():
                fire_gathers(i0 + 2, 0)

            @pl.when(t > 0)
            def _():
                out_copy(i0 - 1, 1).wait()

            process(i0 + 1, 1)
            return carry

        lax.fori_loop(0, n_chunks // 2, step, 0)
        out_copy(n_chunks - 2, 0).wait()
        out_copy(n_chunks - 1, 1).wait()

    return k


def kernel(batch, table):
    R, C = batch.shape
    B = R * C
    flat = batch.reshape(B).astype(jnp.int32)
    half = jnp.zeros((table.shape[0], HPAD), jnp.float32).at[:, :EMB].set(table)
    pair = jnp.concatenate(
        [jnp.repeat(half, table.shape[0], axis=0),
         jnp.tile(half, (table.shape[0], 1))], axis=1)
    out = _make(B)(flat, pair)
    return out.reshape(R, C, EMB)


# speedup vs baseline: 1.4157x; 1.4157x over previous
import functools

import jax
import jax.numpy as jnp
from jax.experimental import pallas as pl
from jax.experimental.pallas import tpu as pltpu

EMB = 50
NUM = 128
BLK = 2048


@functools.cache
def _make_tc(B):
    assert B % BLK == 0
    nb = B // BLK

    def body(idx_ref, tbl_ref, out_ref):
        idx = idx_ref[0, 0, :]
        onehot = (jax.lax.broadcasted_iota(jnp.int32, (BLK, NUM), 1)
                  == idx[:, None]).astype(jnp.float32)
        out_ref[...] = jnp.dot(onehot, tbl_ref[...],
                               preferred_element_type=jnp.float32)

    return pl.pallas_call(
        body,
        grid=(nb,),
        in_specs=[
            pl.BlockSpec((1, 1, BLK), lambda i: (i, 0, 0)),
            pl.BlockSpec((NUM, EMB), lambda i: (0, 0)),
        ],
        out_specs=pl.BlockSpec((BLK, EMB), lambda i: (i, 0)),
        out_shape=jax.ShapeDtypeStruct((B, EMB), jnp.float32),
    )


def kernel(batch, table):
    R, C = batch.shape
    B = R * C
    flat = batch.reshape(B // BLK, 1, BLK).astype(jnp.int32)
    out = _make_tc(B)(flat, table)
    return out.reshape(R, C, EMB)
